# Initial kernel scaffold; baseline (speedup 1.0000x reference)
import jax
import jax.numpy as jnp
from jax.experimental import pallas as pl


def kernel(x, edge_index, W1, b1, W2, b2, W_ih, W_hh, b_ih, b_hh, W_fc, b_fc):
    raise NotImplementedError("write your pallas kernel here")


# speedup vs baseline: 1.5174x; 1.5174x over previous
import functools

import jax
import jax.numpy as jnp
from jax import lax
from jax.experimental import pallas as pl
from jax.experimental.pallas import tpu as pltpu
from jax.experimental.pallas import tpu_sc as plsc

N = 10000
DF = 128
H = 64
E = 320000
LH = 32
NC = 40

FW = 128
NPAD = 10240
RPT = NPAD // 16
CH = 128

MP_TILES = 32
MP_CHUNKS = -(-E // (MP_TILES * CH))
E_MP = MP_TILES * MP_CHUNKS * CH

DEG_TILES = 16
DEG_CHUNKS = -(-E // (DEG_TILES * CH))
E_DEG = DEG_TILES * DEG_CHUNKS * CH

def _deg_body(idx_hbm, out_hbm, hist, idx_v, zbuf, ones_v):
    c = lax.axis_index("c")
    s = lax.axis_index("s")
    w = c * 16 + s
    for i in range(CH // 16):
        ones_v[pl.ds(i * 16, 16)] = jnp.ones((16,), jnp.float32)

    def zfill(i, _):
        zbuf[pl.ds(i * 16, 16)] = jnp.zeros((16,), jnp.float32)
        return 0

    lax.fori_loop(0, RPT // 16, zfill, 0)
    pltpu.sync_copy(zbuf, hist.at[pl.ds(s * RPT, RPT)])
    pltpu.sync_copy(idx_hbm.at[w], idx_v)
    plsc.subcore_barrier()

    def body(j, _):
        pltpu.sync_copy(ones_v, hist.at[idx_v.at[j]], add=True)
        return 0

    lax.fori_loop(0, DEG_CHUNKS, body, 0)
    plsc.subcore_barrier()
    pltpu.sync_copy(hist.at[pl.ds(s * RPT, RPT)],
                    out_hbm.at[pl.ds(c * NPAD + s * RPT, RPT)])


def _mp_body(y_hbm, zeros_hbm, src_hbm, dst_hbm, out_hbm,
             agg, src_v, dst_v, rows_v, sem):
    c = lax.axis_index("c")
    s = lax.axis_index("s")
    w = c * 16 + s
    sl = pl.ds(s * RPT, RPT)
    pltpu.sync_copy(zeros_hbm.at[sl], agg.at[sl])
    pltpu.sync_copy(src_hbm.at[w], src_v)
    pltpu.sync_copy(dst_hbm.at[w], dst_v)
    plsc.subcore_barrier()

    def body(j, _):
        pltpu.async_copy(y_hbm.at[src_v.at[j]], rows_v, sem).wait()
        pltpu.sync_copy(rows_v, agg.at[dst_v.at[j]], add=True)
        return 0

    lax.fori_loop(0, MP_CHUNKS, body, 0)
    plsc.subcore_barrier()
    pltpu.sync_copy(agg.at[sl], out_hbm.at[pl.ds(c * NPAD + s * RPT, RPT)])


@functools.cache
def _sc_kernels():
    mesh = plsc.VectorSubcoreMesh(core_axis_name="c", subcore_axis_name="s")
    deg_k = pl.kernel(
        _deg_body,
        mesh=mesh,
        out_type=jax.ShapeDtypeStruct((2 * NPAD,), jnp.float32),
        scratch_types=[
            pltpu.VMEM_SHARED((NPAD,), jnp.float32),
            pltpu.VMEM((DEG_CHUNKS, CH), jnp.int32),
            pltpu.VMEM((RPT,), jnp.float32),
            pltpu.VMEM((CH,), jnp.float32),
        ],
    )
    mp_k = pl.kernel(
        _mp_body,
        mesh=mesh,
        out_type=jax.ShapeDtypeStruct((2 * NPAD, FW), jnp.float32),
        scratch_types=[
            pltpu.VMEM_SHARED((NPAD, FW), jnp.float32),
            pltpu.VMEM((MP_CHUNKS, CH), jnp.int32),
            pltpu.VMEM((MP_CHUNKS, CH), jnp.int32),
            pltpu.VMEM((CH, FW), jnp.float32),
            pltpu.SemaphoreType.DMA,
        ],
    )
    return deg_k, mp_k


def _norm(d):
    return lax.rsqrt(jnp.maximum(d, 1.0))


def _tc1_body(x_ref, w_ref, ds_ref, o_ref):
    o_ref[...] = jnp.dot(x_ref[...], w_ref[...],
                         preferred_element_type=jnp.float32) * _norm(ds_ref[...])


def _tc2_body(p_ref, dd_ref, ds_ref, b1_ref, w2_ref, o_ref):
    agg = (p_ref[0] + p_ref[1])[:, :H]
    h1 = jax.nn.relu(agg * _norm(dd_ref[...]) + b1_ref[...])
    o_ref[...] = jnp.dot(h1, w2_ref[...],
                         preferred_element_type=jnp.float32) * _norm(ds_ref[...])


def _tc3_body(p_ref, dd_ref, b2_ref, wih_ref, bsum_ref, whh_ref,
              wfc_ref, bfc_ref, o_ref, g_s, hs_s):
    agg = (p_ref[0] + p_ref[1])[:, :H]
    h2 = jax.nn.relu(agg * _norm(dd_ref[...]) + b2_ref[...])
    g_s[...] = jnp.dot(h2, wih_ref[...],
                       preferred_element_type=jnp.float32) + bsum_ref[...]

    lane = lax.broadcasted_iota(jnp.int32, (1, 4 * LH), 1)
    is_tanh = (lane >= 2 * LH) & (lane < 3 * LH)
    gscale = jnp.where(is_tanh, 1.0, 0.5)
    ga = jnp.where(is_tanh, 0.0, 0.5)
    gb = jnp.where(is_tanh, 1.0, 0.5)

    def step(t, carry):
        h, cst = carry
        gates = g_s[pl.ds(t, 1), :] + jnp.dot(
            h, whh_ref[...], preferred_element_type=jnp.float32)
        act = ga + gb * jnp.tanh(gates * gscale)
        i_ = act[:, 0:LH]
        f_ = act[:, LH:2 * LH]
        g_ = act[:, 2 * LH:3 * LH]
        o_ = act[:, 3 * LH:4 * LH]
        cn = f_ * cst + i_ * g_
        hn = o_ * jnp.tanh(cn)
        hs_s[pl.ds(t, 1), :] = hn
        return (hn, cn)

    zero = jnp.zeros((1, LH), jnp.float32)
    lax.fori_loop(0, N, step, (zero, zero))
    o_ref[...] = jnp.dot(hs_s[0:N, :], wfc_ref[...],
                         preferred_element_type=jnp.float32) + bfc_ref[...]


def _pad_idx(idx, total):
    pad = total - E
    padv = (N + (jnp.arange(pad, dtype=jnp.int32) % 64)).astype(jnp.int32)
    return jnp.concatenate([idx, padv])


def kernel(x, edge_index, W1, b1, W2, b2, W_ih, W_hh, b_ih, b_hh, W_fc, b_fc):
    src = edge_index[0].astype(jnp.int32)
    dst = edge_index[1].astype(jnp.int32)

    src_mp = _pad_idx(src, E_MP).reshape(MP_TILES, MP_CHUNKS, CH)
    dst_mp = _pad_idx(dst, E_MP).reshape(MP_TILES, MP_CHUNKS, CH)
    idx_deg = jnp.stack(
        [_pad_idx(src, E_DEG), _pad_idx(dst, E_DEG)]
    ).reshape(2 * DEG_TILES, DEG_CHUNKS, CH)

    _deg_kernel, _mp_kernel = _sc_kernels()
    deg = _deg_kernel(idx_deg).reshape(2, NPAD)
    deg_s = deg[0][:, None]
    deg_d = deg[1][:, None]

    xp = jnp.pad(x, ((0, NPAD - N), (0, 0)))
    zeros_feat = jnp.zeros((NPAD, FW), jnp.float32)
    W1p = jnp.pad(W1, ((0, 0), (0, FW - H)))
    W2p = jnp.pad(W2, ((0, 0), (0, FW - H)))

    y1 = pl.pallas_call(
        _tc1_body,
        grid=(8,),
        in_specs=[
            pl.BlockSpec((NPAD // 8, DF), lambda i: (i, 0)),
            pl.BlockSpec((DF, FW), lambda i: (0, 0)),
            pl.BlockSpec((NPAD // 8, 1), lambda i: (i, 0)),
        ],
        out_specs=pl.BlockSpec((NPAD // 8, FW), lambda i: (i, 0)),
        out_shape=jax.ShapeDtypeStruct((NPAD, FW), jnp.float32),
    )(xp, W1p, deg_s)

    p1 = _mp_kernel(y1, zeros_feat, src_mp, dst_mp).reshape(2, NPAD, FW)

    y2 = pl.pallas_call(
        _tc2_body,
        grid=(8,),
        in_specs=[
            pl.BlockSpec((2, NPAD // 8, FW), lambda i: (0, i, 0)),
            pl.BlockSpec((NPAD // 8, 1), lambda i: (i, 0)),
            pl.BlockSpec((NPAD // 8, 1), lambda i: (i, 0)),
            pl.BlockSpec((1, H), lambda i: (0, 0)),
            pl.BlockSpec((H, FW), lambda i: (0, 0)),
        ],
        out_specs=pl.BlockSpec((NPAD // 8, FW), lambda i: (i, 0)),
        out_shape=jax.ShapeDtypeStruct((NPAD, FW), jnp.float32),
    )(p1, deg_d, deg_s, b1[None, :], W2p)

    p2 = _mp_kernel(y2, zeros_feat, src_mp, dst_mp).reshape(2, NPAD, FW)

    out = pl.pallas_call(
        _tc3_body,
        grid=(1,),
        in_specs=[
            pl.BlockSpec((2, NPAD, FW), lambda i: (0, 0, 0)),
            pl.BlockSpec((NPAD, 1), lambda i: (0, 0)),
            pl.BlockSpec((1, H), lambda i: (0, 0)),
            pl.BlockSpec((H, 4 * LH), lambda i: (0, 0)),
            pl.BlockSpec((1, 4 * LH), lambda i: (0, 0)),
            pl.BlockSpec((LH, 4 * LH), lambda i: (0, 0)),
            pl.BlockSpec((LH, NC), lambda i: (0, 0)),
            pl.BlockSpec((1, NC), lambda i: (0, 0)),
        ],
        out_specs=pl.BlockSpec((N, NC), lambda i: (0, 0)),
        out_shape=jax.ShapeDtypeStruct((N, NC), jnp.float32),
        scratch_shapes=[
            pltpu.VMEM((NPAD, 4 * LH), jnp.float32),
            pltpu.VMEM((NPAD, LH), jnp.float32),
        ],
    )(p2, deg_d, b2[None, :], W_ih.T, (b_ih + b_hh)[None, :], W_hh.T,
      W_fc.T, b_fc[None, :])
    return out


# speedup vs baseline: 3.1193x; 2.0557x over previous
import functools

import jax
import jax.numpy as jnp
from jax import lax
from jax.experimental import pallas as pl
from jax.experimental.pallas import tpu as pltpu
from jax.experimental.pallas import tpu_sc as plsc

N = 10000
DF = 128
H = 64
E = 320000
LH = 32
NC = 40

FW = 128
NPAD = 10240
RPT = NPAD // 16
CH = 128

MP_TILES = 32
MP_CHUNKS = -(-E // (MP_TILES * CH))
E_MP = MP_TILES * MP_CHUNKS * CH

DEG_TILES = 16
DEG_CHUNKS = -(-E // (DEG_TILES * CH))
E_DEG = DEG_TILES * DEG_CHUNKS * CH

def _deg_body(idx_hbm, out_hbm, hist, idx_v, zbuf, ones_v):
    c = lax.axis_index("c")
    s = lax.axis_index("s")
    w = c * 16 + s
    for i in range(CH // 16):
        ones_v[pl.ds(i * 16, 16)] = jnp.ones((16,), jnp.float32)

    def zfill(i, _):
        zbuf[pl.ds(i * 16, 16)] = jnp.zeros((16,), jnp.float32)
        return 0

    lax.fori_loop(0, RPT // 16, zfill, 0)
    pltpu.sync_copy(zbuf, hist.at[pl.ds(s * RPT, RPT)])
    pltpu.sync_copy(idx_hbm.at[w], idx_v)
    plsc.subcore_barrier()

    def body(j, _):
        pltpu.sync_copy(ones_v, hist.at[idx_v.at[j]], add=True)
        return 0

    lax.fori_loop(0, DEG_CHUNKS, body, 0)
    plsc.subcore_barrier()
    pltpu.sync_copy(hist.at[pl.ds(s * RPT, RPT)],
                    out_hbm.at[pl.ds(c * NPAD + s * RPT, RPT)])


def _mp_body(y_hbm, zeros_hbm, src_hbm, dst_hbm, out_hbm,
             agg, src_v, dst_v, rows_v, sem):
    c = lax.axis_index("c")
    s = lax.axis_index("s")
    w = c * 16 + s
    sl = pl.ds(s * RPT, RPT)
    pltpu.sync_copy(zeros_hbm.at[sl], agg.at[sl])
    pltpu.sync_copy(src_hbm.at[w], src_v)
    pltpu.sync_copy(dst_hbm.at[w], dst_v)
    plsc.subcore_barrier()

    def body(j, _):
        pltpu.async_copy(y_hbm.at[src_v.at[j]], rows_v, sem).wait()
        pltpu.sync_copy(rows_v, agg.at[dst_v.at[j]], add=True)
        return 0

    lax.fori_loop(0, MP_CHUNKS, body, 0)
    plsc.subcore_barrier()
    pltpu.sync_copy(agg.at[sl], out_hbm.at[pl.ds(c * NPAD + s * RPT, RPT)])


@functools.cache
def _sc_kernels():
    mesh = plsc.VectorSubcoreMesh(core_axis_name="c", subcore_axis_name="s")
    deg_k = pl.kernel(
        _deg_body,
        mesh=mesh,
        out_type=jax.ShapeDtypeStruct((2 * NPAD,), jnp.float32),
        scratch_types=[
            pltpu.VMEM_SHARED((NPAD,), jnp.float32),
            pltpu.VMEM((DEG_CHUNKS, CH), jnp.int32),
            pltpu.VMEM((RPT,), jnp.float32),
            pltpu.VMEM((CH,), jnp.float32),
        ],
    )
    mp_k = pl.kernel(
        _mp_body,
        mesh=mesh,
        out_type=jax.ShapeDtypeStruct((2 * NPAD, FW), jnp.float32),
        scratch_types=[
            pltpu.VMEM_SHARED((NPAD, FW), jnp.float32),
            pltpu.VMEM((MP_CHUNKS, CH), jnp.int32),
            pltpu.VMEM((MP_CHUNKS, CH), jnp.int32),
            pltpu.VMEM((CH, FW), jnp.float32),
            pltpu.SemaphoreType.DMA,
        ],
    )
    return deg_k, mp_k


def _norm(d):
    return lax.rsqrt(jnp.maximum(d, 1.0))


def _tc1_body(x_ref, w_ref, ds_ref, o_ref):
    o_ref[...] = jnp.dot(x_ref[...], w_ref[...],
                         preferred_element_type=jnp.float32) * _norm(ds_ref[...])


def _tc2_body(p_ref, dd_ref, ds_ref, b1_ref, w2_ref, o_ref):
    agg = (p_ref[0] + p_ref[1])[:, :H]
    h1 = jax.nn.relu(agg * _norm(dd_ref[...]) + b1_ref[...])
    o_ref[...] = jnp.dot(h1, w2_ref[...],
                         preferred_element_type=jnp.float32) * _norm(ds_ref[...])


def _tc3_body(p_ref, dd_ref, b2_ref, wih_ref, bsum_ref, whh_ref,
              wfc_ref, bfc_ref, o_ref, g_s, hs_s):
    agg = (p_ref[0] + p_ref[1])[:, :H]
    h2 = jax.nn.relu(agg * _norm(dd_ref[...]) + b2_ref[...])
    g_s[...] = jnp.dot(h2, wih_ref[...],
                       preferred_element_type=jnp.float32) + bsum_ref[...]

    whh16 = whh_ref[...].astype(jnp.bfloat16)

    def step(t, carry):
        h16, cst = carry
        gates = g_s[pl.ds(t, 1), :] + jnp.dot(
            h16, whh16, preferred_element_type=jnp.float32)
        gi = gates
        gf = pltpu.roll(gates, 3 * LH, 1)
        gg = pltpu.roll(gates, 2 * LH, 1)
        go = pltpu.roll(gates, LH, 1)
        i_ = 0.5 + 0.5 * jnp.tanh(0.5 * gi)
        f_ = 0.5 + 0.5 * jnp.tanh(0.5 * gf)
        g_ = jnp.tanh(gg)
        o_ = 0.5 + 0.5 * jnp.tanh(0.5 * go)
        cn = f_ * cst + i_ * g_
        hn = o_ * jnp.tanh(cn)
        hs_s[pl.ds(t, 1), :] = hn[:, :LH]
        return (hn[:, :LH].astype(jnp.bfloat16), cn)

    lax.fori_loop(0, N, step, (jnp.zeros((1, LH), jnp.bfloat16),
                               jnp.zeros((1, 4 * LH), jnp.float32)))
    o_ref[...] = jnp.dot(hs_s[0:N, :], wfc_ref[...],
                         preferred_element_type=jnp.float32) + bfc_ref[...]


def _pad_idx(idx, total):
    pad = total - E
    padv = (N + (jnp.arange(pad, dtype=jnp.int32) % 64)).astype(jnp.int32)
    return jnp.concatenate([idx, padv])


def kernel(x, edge_index, W1, b1, W2, b2, W_ih, W_hh, b_ih, b_hh, W_fc, b_fc):
    src = edge_index[0].astype(jnp.int32)
    dst = edge_index[1].astype(jnp.int32)

    src_mp = _pad_idx(src, E_MP).reshape(MP_TILES, MP_CHUNKS, CH)
    dst_mp = _pad_idx(dst, E_MP).reshape(MP_TILES, MP_CHUNKS, CH)
    idx_deg = jnp.stack(
        [_pad_idx(src, E_DEG), _pad_idx(dst, E_DEG)]
    ).reshape(2 * DEG_TILES, DEG_CHUNKS, CH)

    _deg_kernel, _mp_kernel = _sc_kernels()
    deg = _deg_kernel(idx_deg).reshape(2, NPAD)
    deg_s = deg[0][:, None]
    deg_d = deg[1][:, None]

    xp = jnp.pad(x, ((0, NPAD - N), (0, 0)))
    zeros_feat = jnp.zeros((NPAD, FW), jnp.float32)
    W1p = jnp.pad(W1, ((0, 0), (0, FW - H)))
    W2p = jnp.pad(W2, ((0, 0), (0, FW - H)))

    y1 = pl.pallas_call(
        _tc1_body,
        grid=(8,),
        in_specs=[
            pl.BlockSpec((NPAD // 8, DF), lambda i: (i, 0)),
            pl.BlockSpec((DF, FW), lambda i: (0, 0)),
            pl.BlockSpec((NPAD // 8, 1), lambda i: (i, 0)),
        ],
        out_specs=pl.BlockSpec((NPAD // 8, FW), lambda i: (i, 0)),
        out_shape=jax.ShapeDtypeStruct((NPAD, FW), jnp.float32),
    )(xp, W1p, deg_s)

    p1 = _mp_kernel(y1, zeros_feat, src_mp, dst_mp).reshape(2, NPAD, FW)

    y2 = pl.pallas_call(
        _tc2_body,
        grid=(8,),
        in_specs=[
            pl.BlockSpec((2, NPAD // 8, FW), lambda i: (0, i, 0)),
            pl.BlockSpec((NPAD // 8, 1), lambda i: (i, 0)),
            pl.BlockSpec((NPAD // 8, 1), lambda i: (i, 0)),
            pl.BlockSpec((1, H), lambda i: (0, 0)),
            pl.BlockSpec((H, FW), lambda i: (0, 0)),
        ],
        out_specs=pl.BlockSpec((NPAD // 8, FW), lambda i: (i, 0)),
        out_shape=jax.ShapeDtypeStruct((NPAD, FW), jnp.float32),
    )(p1, deg_d, deg_s, b1[None, :], W2p)

    p2 = _mp_kernel(y2, zeros_feat, src_mp, dst_mp).reshape(2, NPAD, FW)

    out = pl.pallas_call(
        _tc3_body,
        grid=(1,),
        in_specs=[
            pl.BlockSpec((2, NPAD, FW), lambda i: (0, 0, 0)),
            pl.BlockSpec((NPAD, 1), lambda i: (0, 0)),
            pl.BlockSpec((1, H), lambda i: (0, 0)),
            pl.BlockSpec((H, 4 * LH), lambda i: (0, 0)),
            pl.BlockSpec((1, 4 * LH), lambda i: (0, 0)),
            pl.BlockSpec((LH, 4 * LH), lambda i: (0, 0)),
            pl.BlockSpec((LH, NC), lambda i: (0, 0)),
            pl.BlockSpec((1, NC), lambda i: (0, 0)),
        ],
        out_specs=pl.BlockSpec((N, NC), lambda i: (0, 0)),
        out_shape=jax.ShapeDtypeStruct((N, NC), jnp.float32),
        scratch_shapes=[
            pltpu.VMEM((NPAD, 4 * LH), jnp.float32),
            pltpu.VMEM((NPAD, LH), jnp.float32),
        ],
    )(p2, deg_d, b2[None, :], W_ih.T, (b_ih + b_hh)[None, :], W_hh.T,
      W_fc.T, b_fc[None, :])
    return out


# speedup vs baseline: 4.1460x; 1.3292x over previous
import functools

import jax
import jax.numpy as jnp
from jax import lax
from jax.experimental import pallas as pl
from jax.experimental.pallas import tpu as pltpu
from jax.experimental.pallas import tpu_sc as plsc

N = 10000
DF = 128
H = 64
E = 320000
LH = 32
NC = 40

FW = 128
NPAD = 10240
RPT = NPAD // 16
CH = 128

MP_TILES = 32
MP_CHUNKS = -(-E // (MP_TILES * CH))
E_MP = MP_TILES * MP_CHUNKS * CH

DEG_TILES = 16
DEG_CHUNKS = -(-E // (DEG_TILES * CH))
E_DEG = DEG_TILES * DEG_CHUNKS * CH

def _deg_body(idx_hbm, out_hbm, hist, idx_v, zbuf, ones_v):
    c = lax.axis_index("c")
    s = lax.axis_index("s")
    w = c * 16 + s
    for i in range(CH // 16):
        ones_v[pl.ds(i * 16, 16)] = jnp.ones((16,), jnp.float32)

    def zfill(i, _):
        zbuf[pl.ds(i * 16, 16)] = jnp.zeros((16,), jnp.float32)
        return 0

    lax.fori_loop(0, RPT // 16, zfill, 0)
    pltpu.sync_copy(zbuf, hist.at[pl.ds(s * RPT, RPT)])
    pltpu.sync_copy(idx_hbm.at[w], idx_v)
    plsc.subcore_barrier()

    def body(j, _):
        pltpu.sync_copy(ones_v, hist.at[idx_v.at[j]], add=True)
        return 0

    lax.fori_loop(0, DEG_CHUNKS, body, 0)
    plsc.subcore_barrier()
    pltpu.sync_copy(hist.at[pl.ds(s * RPT, RPT)],
                    out_hbm.at[pl.ds(c * NPAD + s * RPT, RPT)])


def _mp_body(y_hbm, zeros_hbm, src_hbm, dst_hbm, out_hbm,
             agg, src_v, dst_v, rows_v, sem):
    c = lax.axis_index("c")
    s = lax.axis_index("s")
    w = c * 16 + s
    sl = pl.ds(s * RPT, RPT)
    pltpu.sync_copy(zeros_hbm.at[sl], agg.at[sl])
    pltpu.sync_copy(src_hbm.at[w], src_v)
    pltpu.sync_copy(dst_hbm.at[w], dst_v)
    plsc.subcore_barrier()

    def body(j, _):
        pltpu.async_copy(y_hbm.at[src_v.at[j]], rows_v, sem).wait()
        pltpu.sync_copy(rows_v, agg.at[dst_v.at[j]], add=True)
        return 0

    lax.fori_loop(0, MP_CHUNKS, body, 0)
    plsc.subcore_barrier()
    pltpu.sync_copy(agg.at[sl], out_hbm.at[pl.ds(c * NPAD + s * RPT, RPT)])


@functools.cache
def _sc_kernels():
    mesh = plsc.VectorSubcoreMesh(core_axis_name="c", subcore_axis_name="s")
    deg_k = pl.kernel(
        _deg_body,
        mesh=mesh,
        out_type=jax.ShapeDtypeStruct((2 * NPAD,), jnp.float32),
        scratch_types=[
            pltpu.VMEM_SHARED((NPAD,), jnp.float32),
            pltpu.VMEM((DEG_CHUNKS, CH), jnp.int32),
            pltpu.VMEM((RPT,), jnp.float32),
            pltpu.VMEM((CH,), jnp.float32),
        ],
    )
    mp_k = pl.kernel(
        _mp_body,
        mesh=mesh,
        out_type=jax.ShapeDtypeStruct((2 * NPAD, FW), jnp.float32),
        scratch_types=[
            pltpu.VMEM_SHARED((NPAD, FW), jnp.float32),
            pltpu.VMEM((MP_CHUNKS, CH), jnp.int32),
            pltpu.VMEM((MP_CHUNKS, CH), jnp.int32),
            pltpu.VMEM((CH, FW), jnp.float32),
            pltpu.SemaphoreType.DMA,
        ],
    )
    return deg_k, mp_k


def _norm(d):
    return lax.rsqrt(jnp.maximum(d, 1.0))


def _tc1_body(x_ref, w_ref, ds_ref, o_ref):
    o_ref[...] = jnp.dot(x_ref[...], w_ref[...],
                         preferred_element_type=jnp.float32) * _norm(ds_ref[...])


def _tc2_body(p_ref, dd_ref, ds_ref, b1_ref, w2_ref, o_ref):
    agg = (p_ref[0] + p_ref[1])[:, :H]
    h1 = jax.nn.relu(agg * _norm(dd_ref[...]) + b1_ref[...])
    o_ref[...] = jnp.dot(h1, w2_ref[...],
                         preferred_element_type=jnp.float32) * _norm(ds_ref[...])


def _tc3_body(p_ref, dd_ref, b2_ref, wih_ref, bsum_ref, whh_ref,
              wfc_ref, bfc_ref, o_ref, g_s, hs_s):
    agg = (p_ref[0] + p_ref[1])[:, :H]
    h2 = jax.nn.relu(agg * _norm(dd_ref[...]) + b2_ref[...])
    g_s[...] = jnp.dot(h2, wih_ref[...],
                       preferred_element_type=jnp.float32) + bsum_ref[...]

    whh16 = whh_ref[...].astype(jnp.bfloat16)

    def step(t, carry):
        h16, cst = carry
        z = g_s[pl.ds(t, 1), :] + jnp.dot(
            h16, whh16, preferred_element_type=jnp.float32)
        i_ = 0.5 + 0.5 * jnp.tanh(0.5 * z[:, 0:LH])
        f_ = 0.5 + 0.5 * jnp.tanh(0.5 * z[:, 128:128 + LH])
        g_ = jnp.tanh(z[:, 256:256 + LH])
        o_ = 0.5 + 0.5 * jnp.tanh(0.5 * z[:, 384:384 + LH])
        cn = f_ * cst + i_ * g_
        hn = o_ * jnp.tanh(cn)
        hs_s[pl.ds(t, 1), :] = hn
        return (hn.astype(jnp.bfloat16), cn)

    lax.fori_loop(0, N, step, (jnp.zeros((1, LH), jnp.bfloat16),
                               jnp.zeros((1, LH), jnp.float32)))
    o_ref[...] = jnp.dot(hs_s[0:N, :], wfc_ref[...],
                         preferred_element_type=jnp.float32) + bfc_ref[...]


def _pad_idx(idx, total):
    pad = total - E
    padv = (N + (jnp.arange(pad, dtype=jnp.int32) % 64)).astype(jnp.int32)
    return jnp.concatenate([idx, padv])


def kernel(x, edge_index, W1, b1, W2, b2, W_ih, W_hh, b_ih, b_hh, W_fc, b_fc):
    src = edge_index[0].astype(jnp.int32)
    dst = edge_index[1].astype(jnp.int32)

    src_mp = _pad_idx(src, E_MP).reshape(MP_TILES, MP_CHUNKS, CH)
    dst_mp = _pad_idx(dst, E_MP).reshape(MP_TILES, MP_CHUNKS, CH)
    idx_deg = jnp.stack(
        [_pad_idx(src, E_DEG), _pad_idx(dst, E_DEG)]
    ).reshape(2 * DEG_TILES, DEG_CHUNKS, CH)

    _deg_kernel, _mp_kernel = _sc_kernels()
    deg = _deg_kernel(idx_deg).reshape(2, NPAD)
    deg_s = deg[0][:, None]
    deg_d = deg[1][:, None]

    xp = jnp.pad(x, ((0, NPAD - N), (0, 0)))
    zeros_feat = jnp.zeros((NPAD, FW), jnp.float32)
    W1p = jnp.pad(W1, ((0, 0), (0, FW - H)))
    W2p = jnp.pad(W2, ((0, 0), (0, FW - H)))

    y1 = pl.pallas_call(
        _tc1_body,
        grid=(8,),
        in_specs=[
            pl.BlockSpec((NPAD // 8, DF), lambda i: (i, 0)),
            pl.BlockSpec((DF, FW), lambda i: (0, 0)),
            pl.BlockSpec((NPAD // 8, 1), lambda i: (i, 0)),
        ],
        out_specs=pl.BlockSpec((NPAD // 8, FW), lambda i: (i, 0)),
        out_shape=jax.ShapeDtypeStruct((NPAD, FW), jnp.float32),
    )(xp, W1p, deg_s)

    p1 = _mp_kernel(y1, zeros_feat, src_mp, dst_mp).reshape(2, NPAD, FW)

    y2 = pl.pallas_call(
        _tc2_body,
        grid=(8,),
        in_specs=[
            pl.BlockSpec((2, NPAD // 8, FW), lambda i: (0, i, 0)),
            pl.BlockSpec((NPAD // 8, 1), lambda i: (i, 0)),
            pl.BlockSpec((NPAD // 8, 1), lambda i: (i, 0)),
            pl.BlockSpec((1, H), lambda i: (0, 0)),
            pl.BlockSpec((H, FW), lambda i: (0, 0)),
        ],
        out_specs=pl.BlockSpec((NPAD // 8, FW), lambda i: (i, 0)),
        out_shape=jax.ShapeDtypeStruct((NPAD, FW), jnp.float32),
    )(p1, deg_d, deg_s, b1[None, :], W2p)

    p2 = _mp_kernel(y2, zeros_feat, src_mp, dst_mp).reshape(2, NPAD, FW)

    def spread(w):
        blocks = [
            jnp.pad(w[:, k * LH:(k + 1) * LH], ((0, 0), (0, 128 - LH)))
            for k in range(4)
        ]
        return jnp.concatenate(blocks, axis=1)

    wih4 = spread(W_ih.T)
    bsum4 = spread((b_ih + b_hh)[None, :])
    whh4 = spread(W_hh.T)

    out = pl.pallas_call(
        _tc3_body,
        grid=(1,),
        in_specs=[
            pl.BlockSpec((2, NPAD, FW), lambda i: (0, 0, 0)),
            pl.BlockSpec((NPAD, 1), lambda i: (0, 0)),
            pl.BlockSpec((1, H), lambda i: (0, 0)),
            pl.BlockSpec((H, 512), lambda i: (0, 0)),
            pl.BlockSpec((1, 512), lambda i: (0, 0)),
            pl.BlockSpec((LH, 512), lambda i: (0, 0)),
            pl.BlockSpec((LH, NC), lambda i: (0, 0)),
            pl.BlockSpec((1, NC), lambda i: (0, 0)),
        ],
        out_specs=pl.BlockSpec((N, NC), lambda i: (0, 0)),
        out_shape=jax.ShapeDtypeStruct((N, NC), jnp.float32),
        scratch_shapes=[
            pltpu.VMEM((NPAD, 512), jnp.float32),
            pltpu.VMEM((NPAD, LH), jnp.float32),
        ],
    )(p2, deg_d, b2[None, :], wih4, bsum4, whh4,
      W_fc.T, b_fc[None, :])
    return out
